# baseline (speedup 1.0000x reference)
import functools

import jax
import jax.numpy as jnp
from jax import lax
from jax.experimental import pallas as pl
from jax.experimental.pallas import tpu as pltpu
from jax.experimental.pallas import tpu_sc as plsc

_N = 10000
_E = 320000
_D = 128
_CP = 16
_COORDS_RANGE = 10.0

_NC = 2
_NS = 16
_NW = _NC * _NS
_EPW = _E // _NW
_CH = 200
_NCHUNK = _EPW // _CH

_ROWS_PER_TILE = _N // _NS


def _silu(x):
    return x * jax.nn.sigmoid(x)


def _sc_gather(nf, c16, src, dst):
    mesh = plsc.VectorSubcoreMesh(core_axis_name="c", subcore_axis_name="s")

    @functools.partial(
        pl.kernel,
        out_type=(
            jax.ShapeDtypeStruct((_E, _D), jnp.bfloat16),
            jax.ShapeDtypeStruct((_E, _D), jnp.bfloat16),
            jax.ShapeDtypeStruct((_E, _CP), jnp.float32),
            jax.ShapeDtypeStruct((_E, _CP), jnp.float32),
        ),
        mesh=mesh,
        scratch_types=[
            pltpu.VMEM((_CH,), jnp.int32),
            pltpu.VMEM((_CH,), jnp.int32),
            pltpu.VMEM((_CH, _D), jnp.bfloat16),
            pltpu.VMEM((_CH, _D), jnp.bfloat16),
            pltpu.VMEM((_CH, _CP), jnp.float32),
            pltpu.VMEM((_CH, _CP), jnp.float32),
            pltpu.SemaphoreType.DMA,
        ],
        compiler_params=pltpu.CompilerParams(use_tc_tiling_on_sc=False),
    )
    def k(nf_hbm, c16_hbm, src_hbm, dst_hbm,
          gs_hbm, gd_hbm, cs_hbm, cd_hbm,
          idx_s, idx_d, buf_s, buf_d, buf_cs, buf_cd, sem):
        c = lax.axis_index("c")
        s = lax.axis_index("s")
        wid = s * _NC + c
        base0 = wid * _EPW

        def chunk(j, carry):
            base = base0 + j * _CH
            pltpu.sync_copy(src_hbm.at[pl.ds(base, _CH)], idx_s)
            pltpu.sync_copy(dst_hbm.at[pl.ds(base, _CH)], idx_d)
            a1 = pltpu.async_copy(nf_hbm.at[idx_s], buf_s, sem)
            a2 = pltpu.async_copy(nf_hbm.at[idx_d], buf_d, sem)
            a3 = pltpu.async_copy(c16_hbm.at[idx_s], buf_cs, sem)
            a4 = pltpu.async_copy(c16_hbm.at[idx_d], buf_cd, sem)
            a1.wait()
            a2.wait()
            a3.wait()
            a4.wait()
            pltpu.sync_copy(buf_s, gs_hbm.at[pl.ds(base, _CH)])
            pltpu.sync_copy(buf_d, gd_hbm.at[pl.ds(base, _CH)])
            pltpu.sync_copy(buf_cs, cs_hbm.at[pl.ds(base, _CH)])
            pltpu.sync_copy(buf_cd, cd_hbm.at[pl.ds(base, _CH)])
            return carry

        lax.fori_loop(0, _NCHUNK, chunk, 0)

    return k(nf, c16, src, dst)


_BE = 512


def _edge_body(nfs_ref, nfd_ref, cs_ref, cd_ref, ef_ref,
               Ws_ref, Wd_ref, Wec_ref, wr_ref, b1_ref,
               We2_ref, be2_ref, wa_ref, ba_ref, wco_ref,
               mh_ref, mx_ref):
    d = cs_ref[...] - cd_ref[...]
    r2 = jnp.sum(d * d, axis=1, keepdims=True)
    radial = jnp.sqrt(r2 + 1e-12)
    xdiff = d / (radial + 1.0)

    pre = (jnp.dot(nfs_ref[...], Ws_ref[...], preferred_element_type=jnp.float32)
           + jnp.dot(nfd_ref[...], Wd_ref[...], preferred_element_type=jnp.float32)
           + jnp.dot(ef_ref[...], Wec_ref[...], preferred_element_type=jnp.float32)
           + radial * wr_ref[...]
           + b1_ref[...])
    h1 = _silu(pre[:, :_D])
    c1 = _silu(pre[:, _D:])
    m = _silu(jnp.dot(h1.astype(jnp.bfloat16), We2_ref[...],
                      preferred_element_type=jnp.float32)
              + be2_ref[...])
    att = jax.nn.sigmoid(
        jnp.sum(m * wa_ref[...], axis=1, keepdims=True) + ba_ref[...])
    mh_ref[...] = m * att
    cc = jnp.sum(c1 * wco_ref[...], axis=1, keepdims=True)
    mx_ref[...] = jnp.tanh(cc) * xdiff * _COORDS_RANGE


def _tc_edge(nfs, nfd, cs, cd, ef, Ws, Wd, Wec, wr, b1, We2, be2, wa, ba, wco):
    nblk = _E // _BE
    full = lambda r, c_: pl.BlockSpec((r, c_), lambda i: (0, 0))
    blk = lambda c_: pl.BlockSpec((_BE, c_), lambda i: (i, 0))
    return pl.pallas_call(
        _edge_body,
        grid=(nblk,),
        in_specs=[
            blk(_D), blk(_D), blk(_CP), blk(_CP), blk(16),
            full(_D, 256), full(_D, 256), full(16, 256), full(1, 256),
            full(1, 256), full(_D, _D), full(1, _D), full(1, _D),
            full(1, 1), full(1, _D),
        ],
        out_specs=[blk(_D), blk(_CP)],
        out_shape=(
            jax.ShapeDtypeStruct((_E, _D), jnp.float32),
            jax.ShapeDtypeStruct((_E, _CP), jnp.float32),
        ),
    )(nfs, nfd, cs, cd, ef, Ws, Wd, Wec, wr, b1, We2, be2, wa, ba, wco)


def _sc_scatter(mh, mx, dst, zh, zx):
    mesh = plsc.VectorSubcoreMesh(core_axis_name="c", subcore_axis_name="s")

    @functools.partial(
        pl.kernel,
        out_type=(
            jax.ShapeDtypeStruct((_NC, _N, _D), jnp.float32),
            jax.ShapeDtypeStruct((_NC, _N, _CP), jnp.float32),
        ),
        mesh=mesh,
        scratch_types=[
            pltpu.VMEM_SHARED((_N, _D), jnp.float32),
            pltpu.VMEM_SHARED((_N, _CP), jnp.float32),
            pltpu.VMEM((_CH,), jnp.int32),
            pltpu.VMEM((_CH, _D), jnp.float32),
            pltpu.VMEM((_CH, _CP), jnp.float32),
        ],
        compiler_params=pltpu.CompilerParams(use_tc_tiling_on_sc=False),
    )
    def k(mh_hbm, mx_hbm, dst_hbm, zh_hbm, zx_hbm,
          ph_hbm, px_hbm,
          h_acc, x_acc, idx_v, buf_h, buf_x):
        c = lax.axis_index("c")
        s = lax.axis_index("s")
        wid = s * _NC + c
        base0 = wid * _EPW
        row0 = s * _ROWS_PER_TILE

        pltpu.sync_copy(zh_hbm.at[pl.ds(row0, _ROWS_PER_TILE)],
                        h_acc.at[pl.ds(row0, _ROWS_PER_TILE)])
        pltpu.sync_copy(zx_hbm.at[pl.ds(row0, _ROWS_PER_TILE)],
                        x_acc.at[pl.ds(row0, _ROWS_PER_TILE)])
        plsc.subcore_barrier()

        def chunk(j, carry):
            base = base0 + j * _CH
            pltpu.sync_copy(dst_hbm.at[pl.ds(base, _CH)], idx_v)
            pltpu.sync_copy(mh_hbm.at[pl.ds(base, _CH)], buf_h)
            pltpu.sync_copy(mx_hbm.at[pl.ds(base, _CH)], buf_x)
            pltpu.sync_copy(buf_h, h_acc.at[idx_v], add=True)
            pltpu.sync_copy(buf_x, x_acc.at[idx_v], add=True)
            return carry

        lax.fori_loop(0, _NCHUNK, chunk, 0)
        plsc.subcore_barrier()

        pltpu.sync_copy(h_acc.at[pl.ds(row0, _ROWS_PER_TILE)],
                        ph_hbm.at[c].at[pl.ds(row0, _ROWS_PER_TILE)])
        pltpu.sync_copy(x_acc.at[pl.ds(row0, _ROWS_PER_TILE)],
                        px_hbm.at[c].at[pl.ds(row0, _ROWS_PER_TILE)])

    return k(mh, mx, dst, zh, zx)


_BN = 1000


def _node_body(nf_ref, c16_ref, z_ref, ph0_ref, ph1_ref, px0_ref, px1_ref,
               Wn1a_ref, Wn1b_ref, bn1_ref, Wn2_ref, bn2_ref, g_ref, b_ref,
               h_ref, x_ref):
    zinv = 1.0 / z_ref[...]
    hn = (ph0_ref[...] + ph1_ref[...]) * zinv
    xn = (px0_ref[...] + px1_ref[...]) * zinv
    t = _silu(jnp.dot(nf_ref[...].astype(jnp.bfloat16), Wn1a_ref[...],
                      preferred_element_type=jnp.float32)
              + jnp.dot(hn.astype(jnp.bfloat16), Wn1b_ref[...],
                        preferred_element_type=jnp.float32)
              + bn1_ref[...])
    h = jnp.dot(t.astype(jnp.bfloat16), Wn2_ref[...],
                preferred_element_type=jnp.float32) + bn2_ref[...]
    mu = jnp.mean(h, axis=1, keepdims=True)
    var = jnp.mean((h - mu) * (h - mu), axis=1, keepdims=True)
    h_ref[...] = (h - mu) / jnp.sqrt(var + 1e-5) * g_ref[...] + b_ref[...]
    x_ref[...] = c16_ref[...] + xn


def _tc_node(nf, c16, z, ph0, ph1, px0, px1, Wn1a, Wn1b, bn1, Wn2, bn2, g, b):
    nblk = _N // _BN
    full = lambda r, c_: pl.BlockSpec((r, c_), lambda i: (0, 0))
    blk = lambda c_: pl.BlockSpec((_BN, c_), lambda i: (i, 0))
    return pl.pallas_call(
        _node_body,
        grid=(nblk,),
        in_specs=[
            blk(_D), blk(_CP), blk(1), blk(_D), blk(_D), blk(_CP), blk(_CP),
            full(_D, _D), full(_D, _D), full(1, _D), full(_D, _D),
            full(1, _D), full(1, _D), full(1, _D),
        ],
        out_specs=[blk(_D), blk(_CP)],
        out_shape=(
            jax.ShapeDtypeStruct((_N, _D), jnp.float32),
            jax.ShapeDtypeStruct((_N, _CP), jnp.float32),
        ),
    )(nf, c16, z, ph0, ph1, px0, px1, Wn1a, Wn1b, bn1, Wn2, bn2, g, b)


def kernel(node_feat, coord_feat, z, edge_feat, edge_index,
           We1, be1, We2, be2, Wa, ba, Wc1, bc1, Wc_out,
           Wn1, bn1, Wn2, bn2, ln_g, ln_b):
    src = edge_index[0].astype(jnp.int32)
    dst = edge_index[1].astype(jnp.int32)
    c16 = jnp.pad(coord_feat, ((0, 0), (0, _CP - 3)))

    Ws = jnp.concatenate([We1[:_D], Wc1[:_D]], axis=1).astype(jnp.bfloat16)
    Wd = jnp.concatenate([We1[_D:2 * _D], Wc1[_D:2 * _D]], axis=1).astype(jnp.bfloat16)
    Wec = jnp.concatenate([We1[2 * _D + 1:], Wc1[2 * _D + 1:]],
                          axis=1).astype(jnp.bfloat16)
    wr = jnp.concatenate([We1[2 * _D], Wc1[2 * _D]])[None, :]
    b1 = jnp.concatenate([be1, bc1])[None, :]
    be2r = be2[None, :]
    wa = Wa[:, 0][None, :]
    bar = ba.reshape(1, 1)
    wco = Wc_out[:, 0][None, :]
    Wn1a = Wn1[:_D].astype(jnp.bfloat16)
    Wn1b = Wn1[_D:].astype(jnp.bfloat16)
    Wn2b = Wn2.astype(jnp.bfloat16)
    bn1r = bn1[None, :]
    bn2r = bn2[None, :]
    gr = ln_g[None, :]
    br = ln_b[None, :]

    gs, gd, cs, cd = _sc_gather(node_feat.astype(jnp.bfloat16), c16, src, dst)
    mh, mx = _tc_edge(gs, gd, cs, cd, edge_feat.astype(jnp.bfloat16),
                      Ws, Wd, Wec, wr, b1, We2.astype(jnp.bfloat16),
                      be2r, wa, bar, wco)
    zh = jnp.zeros((_N, _D), jnp.float32)
    zx = jnp.zeros((_N, _CP), jnp.float32)
    ph, px = _sc_scatter(mh, mx, dst, zh, zx)
    h, x16 = _tc_node(node_feat, c16, z, ph[0], ph[1], px[0], px[1],
                      Wn1a, Wn1b, bn1r, Wn2b, bn2r, gr, br)
    return (h, x16[:, :3])


# speedup vs baseline: 1.3150x; 1.3150x over previous
import functools

import jax
import jax.numpy as jnp
from jax import lax
from jax.experimental import pallas as pl
from jax.experimental.pallas import tpu as pltpu
from jax.experimental.pallas import tpu_sc as plsc

_N = 10000
_E = 320000
_D = 128
_CP = 16
_COORDS_RANGE = 10.0

_NC = 2
_NS = 16
_NW = _NC * _NS
_EPW = _E // _NW
_CH = 200
_NCHUNK = _EPW // _CH

_ROWS_PER_TILE = _N // _NS


def _silu(x):
    return x * jax.nn.sigmoid(x)


def _sc_gather(nf, c16, src, dst):
    mesh = plsc.VectorSubcoreMesh(core_axis_name="c", subcore_axis_name="s")

    @functools.partial(
        pl.kernel,
        out_type=(
            jax.ShapeDtypeStruct((_E, _D), jnp.float32),
            jax.ShapeDtypeStruct((_E, _D), jnp.float32),
            jax.ShapeDtypeStruct((_E, _CP), jnp.float32),
            jax.ShapeDtypeStruct((_E, _CP), jnp.float32),
        ),
        mesh=mesh,
        scratch_types=[
            pltpu.VMEM((_CH,), jnp.int32),
            pltpu.VMEM((_CH,), jnp.int32),
            pltpu.VMEM((_CH, _D), jnp.float32),
            pltpu.VMEM((_CH, _D), jnp.float32),
            pltpu.VMEM((_CH, _CP), jnp.float32),
            pltpu.VMEM((_CH, _CP), jnp.float32),
            pltpu.SemaphoreType.DMA,
        ],
        compiler_params=pltpu.CompilerParams(use_tc_tiling_on_sc=False),
    )
    def k(nf_hbm, c16_hbm, src_hbm, dst_hbm,
          gs_hbm, gd_hbm, cs_hbm, cd_hbm,
          idx_s, idx_d, buf_s, buf_d, buf_cs, buf_cd, sem):
        c = lax.axis_index("c")
        s = lax.axis_index("s")
        wid = s * _NC + c
        base0 = wid * _EPW

        def chunk(j, carry):
            base = base0 + j * _CH
            pltpu.sync_copy(src_hbm.at[pl.ds(base, _CH)], idx_s)
            pltpu.sync_copy(dst_hbm.at[pl.ds(base, _CH)], idx_d)
            a1 = pltpu.async_copy(nf_hbm.at[idx_s], buf_s, sem)
            a2 = pltpu.async_copy(nf_hbm.at[idx_d], buf_d, sem)
            a3 = pltpu.async_copy(c16_hbm.at[idx_s], buf_cs, sem)
            a4 = pltpu.async_copy(c16_hbm.at[idx_d], buf_cd, sem)
            a1.wait()
            a2.wait()
            a3.wait()
            a4.wait()
            pltpu.sync_copy(buf_s, gs_hbm.at[pl.ds(base, _CH)])
            pltpu.sync_copy(buf_d, gd_hbm.at[pl.ds(base, _CH)])
            pltpu.sync_copy(buf_cs, cs_hbm.at[pl.ds(base, _CH)])
            pltpu.sync_copy(buf_cd, cd_hbm.at[pl.ds(base, _CH)])
            return carry

        lax.fori_loop(0, _NCHUNK, chunk, 0)

    return k(nf, c16, src, dst)


_BE = 512


def _edge_body(nfs_ref, nfd_ref, cs_ref, cd_ref, ef_ref,
               Ws_ref, Wd_ref, Wec_ref, wr_ref, b1_ref,
               We2_ref, be2_ref, wa_ref, ba_ref, wco_ref,
               mh_ref, mx_ref):
    d = cs_ref[...] - cd_ref[...]
    r2 = jnp.sum(d * d, axis=1, keepdims=True)
    radial = jnp.sqrt(r2 + 1e-12)
    xdiff = d / (radial + 1.0)

    pre = (jnp.dot(nfs_ref[...].astype(jnp.bfloat16), Ws_ref[...],
                   preferred_element_type=jnp.float32)
           + jnp.dot(nfd_ref[...].astype(jnp.bfloat16), Wd_ref[...],
                     preferred_element_type=jnp.float32)
           + jnp.dot(ef_ref[...], Wec_ref[...], preferred_element_type=jnp.float32)
           + radial * wr_ref[...]
           + b1_ref[...])
    h1 = _silu(pre[:, :_D])
    c1 = _silu(pre[:, _D:])
    m = _silu(jnp.dot(h1.astype(jnp.bfloat16), We2_ref[...],
                      preferred_element_type=jnp.float32)
              + be2_ref[...])
    att = jax.nn.sigmoid(
        jnp.sum(m * wa_ref[...], axis=1, keepdims=True) + ba_ref[...])
    mh_ref[...] = m * att
    cc = jnp.sum(c1 * wco_ref[...], axis=1, keepdims=True)
    mx_ref[...] = jnp.tanh(cc) * xdiff * _COORDS_RANGE


def _tc_edge(nfs, nfd, cs, cd, ef, Ws, Wd, Wec, wr, b1, We2, be2, wa, ba, wco):
    nblk = _E // _BE
    full = lambda r, c_: pl.BlockSpec((r, c_), lambda i: (0, 0))
    blk = lambda c_: pl.BlockSpec((_BE, c_), lambda i: (i, 0))
    return pl.pallas_call(
        _edge_body,
        grid=(nblk,),
        in_specs=[
            blk(_D), blk(_D), blk(_CP), blk(_CP), blk(16),
            full(_D, 256), full(_D, 256), full(16, 256), full(1, 256),
            full(1, 256), full(_D, _D), full(1, _D), full(1, _D),
            full(1, 1), full(1, _D),
        ],
        out_specs=[blk(_D), blk(_CP)],
        out_shape=(
            jax.ShapeDtypeStruct((_E, _D), jnp.float32),
            jax.ShapeDtypeStruct((_E, _CP), jnp.float32),
        ),
    )(nfs, nfd, cs, cd, ef, Ws, Wd, Wec, wr, b1, We2, be2, wa, ba, wco)


def _sc_scatter(mh, mx, dst, zh, zx):
    mesh = plsc.VectorSubcoreMesh(core_axis_name="c", subcore_axis_name="s")

    @functools.partial(
        pl.kernel,
        out_type=(
            jax.ShapeDtypeStruct((_NC, _N, _D), jnp.float32),
            jax.ShapeDtypeStruct((_NC, _N, _CP), jnp.float32),
        ),
        mesh=mesh,
        scratch_types=[
            pltpu.VMEM_SHARED((_N, _D), jnp.float32),
            pltpu.VMEM_SHARED((_N, _CP), jnp.float32),
            pltpu.VMEM((_CH,), jnp.int32),
            pltpu.VMEM((_CH, _D), jnp.float32),
            pltpu.VMEM((_CH, _CP), jnp.float32),
        ],
        compiler_params=pltpu.CompilerParams(use_tc_tiling_on_sc=False),
    )
    def k(mh_hbm, mx_hbm, dst_hbm, zh_hbm, zx_hbm,
          ph_hbm, px_hbm,
          h_acc, x_acc, idx_v, buf_h, buf_x):
        c = lax.axis_index("c")
        s = lax.axis_index("s")
        wid = s * _NC + c
        base0 = wid * _EPW
        row0 = s * _ROWS_PER_TILE

        pltpu.sync_copy(zh_hbm.at[pl.ds(row0, _ROWS_PER_TILE)],
                        h_acc.at[pl.ds(row0, _ROWS_PER_TILE)])
        pltpu.sync_copy(zx_hbm.at[pl.ds(row0, _ROWS_PER_TILE)],
                        x_acc.at[pl.ds(row0, _ROWS_PER_TILE)])
        plsc.subcore_barrier()

        def chunk(j, carry):
            base = base0 + j * _CH
            pltpu.sync_copy(dst_hbm.at[pl.ds(base, _CH)], idx_v)
            pltpu.sync_copy(mh_hbm.at[pl.ds(base, _CH)], buf_h)
            pltpu.sync_copy(mx_hbm.at[pl.ds(base, _CH)], buf_x)
            pltpu.sync_copy(buf_h, h_acc.at[idx_v], add=True)
            pltpu.sync_copy(buf_x, x_acc.at[idx_v], add=True)
            return carry

        lax.fori_loop(0, _NCHUNK, chunk, 0)
        plsc.subcore_barrier()

        pltpu.sync_copy(h_acc.at[pl.ds(row0, _ROWS_PER_TILE)],
                        ph_hbm.at[c].at[pl.ds(row0, _ROWS_PER_TILE)])
        pltpu.sync_copy(x_acc.at[pl.ds(row0, _ROWS_PER_TILE)],
                        px_hbm.at[c].at[pl.ds(row0, _ROWS_PER_TILE)])

    return k(mh, mx, dst, zh, zx)


_BN = 1000


def _node_body(nf_ref, c16_ref, z_ref, ph0_ref, ph1_ref, px0_ref, px1_ref,
               Wn1a_ref, Wn1b_ref, bn1_ref, Wn2_ref, bn2_ref, g_ref, b_ref,
               h_ref, x_ref):
    zinv = 1.0 / z_ref[...]
    hn = (ph0_ref[...] + ph1_ref[...]) * zinv
    xn = (px0_ref[...] + px1_ref[...]) * zinv
    t = _silu(jnp.dot(nf_ref[...].astype(jnp.bfloat16), Wn1a_ref[...],
                      preferred_element_type=jnp.float32)
              + jnp.dot(hn.astype(jnp.bfloat16), Wn1b_ref[...],
                        preferred_element_type=jnp.float32)
              + bn1_ref[...])
    h = jnp.dot(t.astype(jnp.bfloat16), Wn2_ref[...],
                preferred_element_type=jnp.float32) + bn2_ref[...]
    mu = jnp.mean(h, axis=1, keepdims=True)
    var = jnp.mean((h - mu) * (h - mu), axis=1, keepdims=True)
    h_ref[...] = (h - mu) / jnp.sqrt(var + 1e-5) * g_ref[...] + b_ref[...]
    x_ref[...] = c16_ref[...] + xn


def _tc_node(nf, c16, z, ph0, ph1, px0, px1, Wn1a, Wn1b, bn1, Wn2, bn2, g, b):
    nblk = _N // _BN
    full = lambda r, c_: pl.BlockSpec((r, c_), lambda i: (0, 0))
    blk = lambda c_: pl.BlockSpec((_BN, c_), lambda i: (i, 0))
    return pl.pallas_call(
        _node_body,
        grid=(nblk,),
        in_specs=[
            blk(_D), blk(_CP), blk(1), blk(_D), blk(_D), blk(_CP), blk(_CP),
            full(_D, _D), full(_D, _D), full(1, _D), full(_D, _D),
            full(1, _D), full(1, _D), full(1, _D),
        ],
        out_specs=[blk(_D), blk(_CP)],
        out_shape=(
            jax.ShapeDtypeStruct((_N, _D), jnp.float32),
            jax.ShapeDtypeStruct((_N, _CP), jnp.float32),
        ),
    )(nf, c16, z, ph0, ph1, px0, px1, Wn1a, Wn1b, bn1, Wn2, bn2, g, b)


def kernel(node_feat, coord_feat, z, edge_feat, edge_index,
           We1, be1, We2, be2, Wa, ba, Wc1, bc1, Wc_out,
           Wn1, bn1, Wn2, bn2, ln_g, ln_b):
    src = edge_index[0].astype(jnp.int32)
    dst = edge_index[1].astype(jnp.int32)
    c16 = jnp.pad(coord_feat, ((0, 0), (0, _CP - 3)))

    Ws = jnp.concatenate([We1[:_D], Wc1[:_D]], axis=1).astype(jnp.bfloat16)
    Wd = jnp.concatenate([We1[_D:2 * _D], Wc1[_D:2 * _D]], axis=1).astype(jnp.bfloat16)
    Wec = jnp.concatenate([We1[2 * _D + 1:], Wc1[2 * _D + 1:]],
                          axis=1).astype(jnp.bfloat16)
    wr = jnp.concatenate([We1[2 * _D], Wc1[2 * _D]])[None, :]
    b1 = jnp.concatenate([be1, bc1])[None, :]
    be2r = be2[None, :]
    wa = Wa[:, 0][None, :]
    bar = ba.reshape(1, 1)
    wco = Wc_out[:, 0][None, :]
    Wn1a = Wn1[:_D].astype(jnp.bfloat16)
    Wn1b = Wn1[_D:].astype(jnp.bfloat16)
    Wn2b = Wn2.astype(jnp.bfloat16)
    bn1r = bn1[None, :]
    bn2r = bn2[None, :]
    gr = ln_g[None, :]
    br = ln_b[None, :]

    gs, gd, cs, cd = _sc_gather(node_feat, c16, src, dst)
    mh, mx = _tc_edge(gs, gd, cs, cd, edge_feat.astype(jnp.bfloat16),
                      Ws, Wd, Wec, wr, b1, We2.astype(jnp.bfloat16),
                      be2r, wa, bar, wco)
    zh = jnp.zeros((_N, _D), jnp.float32)
    zx = jnp.zeros((_N, _CP), jnp.float32)
    ph, px = _sc_scatter(mh, mx, dst, zh, zx)
    h, x16 = _tc_node(node_feat, c16, z, ph[0], ph[1], px[0], px[1],
                      Wn1a, Wn1b, bn1r, Wn2b, bn2r, gr, br)
    return (h, x16[:, :3])
